# baseline (speedup 1.0000x reference)
import functools

import numpy as np

import jax
import jax.numpy as jnp
from jax.experimental import pallas as pl
from jax.experimental.pallas import tpu as pltpu

B, CIN, COUT, T, KW = 16, 64, 64, 4096, 64
FQ = 1024
TOUT = 2 * (FQ - 1)
TPAD = 2048

OB = 8
FB = 512
NSTEP = COUT // OB


@functools.lru_cache(maxsize=1)
def _bases():
    t = np.arange(T, dtype=np.float64)[:, None]
    f = np.arange(FQ, dtype=np.float64)[None, :]
    ang = 2.0 * np.pi * t * f / T
    fwd = np.concatenate([np.cos(ang), -np.sin(ang)], axis=1)

    k = np.arange(KW, dtype=np.float64)[:, None]
    angw = 2.0 * np.pi * (T - KW + k) * f / T
    fwdw = np.concatenate(
        [np.cos(angw), -np.sin(angw), np.cos(angw) - np.sin(angw)], axis=1)

    tt = np.arange(TOUT, dtype=np.float64)[None, :]
    fi = np.arange(FQ, dtype=np.float64)[:, None]
    angi = 2.0 * np.pi * fi * tt / TOUT
    alpha = np.full((FQ, 1), 2.0)
    alpha[0, 0] = 1.0
    alpha[FQ - 1, 0] = 1.0
    ar = alpha * np.cos(angi) / TOUT
    ai = -2.0 * np.sin(angi) / TOUT
    ai[0, :] = 0.0
    ai[FQ - 1, :] = 0.0
    ar = np.pad(ar, ((0, 0), (0, TPAD - TOUT)))
    ai = np.pad(ai, ((0, 0), (0, TPAD - TOUT)))
    return (fwd.astype(np.float32), fwdw.astype(np.float32),
            ar.astype(np.float32), ai.astype(np.float32))


def _matmul_kernel(x_ref, w_ref, o_ref):
    o_ref[...] = jnp.dot(x_ref[...], w_ref[...],
                         preferred_element_type=jnp.float32)


def _mm(x, w, bm, bn):
    m, k = x.shape
    _, n = w.shape
    return pl.pallas_call(
        _matmul_kernel,
        grid=(m // bm, n // bn),
        in_specs=[
            pl.BlockSpec((bm, k), lambda i, j: (i, 0)),
            pl.BlockSpec((k, bn), lambda i, j: (0, j)),
        ],
        out_specs=pl.BlockSpec((bm, bn), lambda i, j: (i, j)),
        out_shape=jax.ShapeDtypeStruct((m, n), jnp.float32),
        compiler_params=pltpu.CompilerParams(
            dimension_semantics=("parallel", "arbitrary"),
        ),
    )(x, w)


def _fused_kernel(w2_ref, ew_ref, x_ref, ar_ref, ai_ref, bias_ref, o_ref,
                  wdft_ref, cr_ref, ci_ref):
    s = pl.program_id(0)
    buf = jax.lax.rem(s, 2)
    pbuf = jax.lax.rem(s + 1, 2)

    wdft_ref[:, 0, :] = jnp.dot(w2_ref[...].astype(jnp.bfloat16), ew_ref[...],
                                preferred_element_type=jnp.float32)

    for pj in range(OB // 2):
        for half in range(FQ // FB):
            lo = half * FB
            accs = [None] * 6
            for c in range(CIN):
                xr = x_ref[c, :, lo:lo + FB]
                xi = x_ref[c, :, FQ + lo:FQ + lo + FB]
                xs = xr + xi
                for u in range(2):
                    r = (2 * pj + u) * CIN + c
                    m1 = xr * wdft_ref[r, :, lo:lo + FB]
                    m2 = xi * wdft_ref[r, :, FQ + lo:FQ + lo + FB]
                    m3 = xs * wdft_ref[r, :, 2 * FQ + lo:2 * FQ + lo + FB]
                    j = 3 * u
                    accs[j] = m1 if accs[j] is None else accs[j] + m1
                    accs[j + 1] = m2 if accs[j + 1] is None else accs[j + 1] + m2
                    accs[j + 2] = m3 if accs[j + 2] is None else accs[j + 2] + m3
            for u in range(2):
                a1, a2, a3 = accs[3 * u:3 * u + 3]
                cr_ref[buf, 2 * pj + u, :, lo:lo + FB] = (
                    (a1 - a2).astype(cr_ref.dtype))
                ci_ref[buf, 2 * pj + u, :, lo:lo + FB] = (
                    (a3 - a1 - a2).astype(ci_ref.dtype))

    cr = cr_ref[pbuf].reshape(OB * B, FQ)
    ci = ci_ref[pbuf].reshape(OB * B, FQ)
    acc = jnp.dot(cr, ar_ref[...], preferred_element_type=jnp.float32)
    acc += jnp.dot(ci, ai_ref[...], preferred_element_type=jnp.float32)
    o_ref[...] = acc + pltpu.repeat(bias_ref[...], TPAD // 128, axis=1)


def _fused(w2, ew, xhat, inv_r, inv_i, bias_plane):
    xv = xhat.reshape(CIN, B, 2 * FQ)
    mrows = OB * B
    return pl.pallas_call(
        _fused_kernel,
        grid=(NSTEP + 1,),
        in_specs=[
            pl.BlockSpec((OB * CIN, KW),
                         lambda s: (jnp.minimum(s, NSTEP - 1), 0)),
            pl.BlockSpec((KW, 3 * FQ), lambda s: (0, 0)),
            pl.BlockSpec((CIN, B, 2 * FQ), lambda s: (0, 0, 0)),
            pl.BlockSpec((FQ, TPAD), lambda s: (0, 0)),
            pl.BlockSpec((FQ, TPAD), lambda s: (0, 0)),
            pl.BlockSpec((mrows, 128), lambda s: (jnp.maximum(s - 1, 0), 0)),
        ],
        out_specs=pl.BlockSpec((mrows, TPAD),
                               lambda s: (jnp.maximum(s - 1, 0), 0)),
        out_shape=jax.ShapeDtypeStruct((COUT * B, TPAD), jnp.float32),
        scratch_shapes=[
            pltpu.VMEM((OB * CIN, 1, 3 * FQ), jnp.float32),
            pltpu.VMEM((2, OB, B, FQ), jnp.bfloat16),
            pltpu.VMEM((2, OB, B, FQ), jnp.bfloat16),
        ],
        compiler_params=pltpu.CompilerParams(
            dimension_semantics=("arbitrary",),
            vmem_limit_bytes=100 * 1024 * 1024,
        ),
    )(w2, ew, xv, inv_r, inv_i, bias_plane)


def kernel(x, weight, bias):
    fwd, fwdw, inv_r, inv_i = _bases()
    fwd = jnp.asarray(fwd, jnp.bfloat16)
    fwdw = jnp.asarray(fwdw, jnp.bfloat16)
    inv_r = jnp.asarray(inv_r, jnp.bfloat16)
    inv_i = jnp.asarray(inv_i, jnp.bfloat16)

    xt = x.transpose(1, 0, 2).reshape(CIN * B, T).astype(jnp.bfloat16)
    xhat = _mm(xt, fwd, bm=512, bn=1024)

    w2 = weight.reshape(COUT * CIN, KW)
    bias_plane = jnp.broadcast_to(
        jnp.repeat(bias, B)[:, None], (COUT * B, 128)).astype(jnp.float32)

    out = _fused(w2, fwdw, xhat, inv_r, inv_i, bias_plane)

    return out[:, :TOUT].reshape(COUT, B, TOUT).transpose(1, 0, 2)


# speedup vs baseline: 1.0378x; 1.0378x over previous
import functools

import numpy as np

import jax
import jax.numpy as jnp
from jax.experimental import pallas as pl
from jax.experimental.pallas import tpu as pltpu

B, CIN, COUT, T, KW = 16, 64, 64, 4096, 64
FQ = 1024
TOUT = 2 * (FQ - 1)
TPAD = 2048

OB = 8
FB = 512
NSTEP = COUT // OB


@functools.lru_cache(maxsize=1)
def _bases():
    t = np.arange(T, dtype=np.float64)[:, None]
    f = np.arange(FQ, dtype=np.float64)[None, :]
    ang = 2.0 * np.pi * t * f / T
    fwd = np.concatenate([np.cos(ang), -np.sin(ang)], axis=1)

    k = np.arange(KW, dtype=np.float64)[:, None]
    angw = 2.0 * np.pi * (T - KW + k) * f / T
    fwdw = np.concatenate(
        [np.cos(angw), -np.sin(angw), np.cos(angw) - np.sin(angw)], axis=1)

    tt = np.arange(TOUT, dtype=np.float64)[None, :]
    fi = np.arange(FQ, dtype=np.float64)[:, None]
    angi = 2.0 * np.pi * fi * tt / TOUT
    alpha = np.full((FQ, 1), 2.0)
    alpha[0, 0] = 1.0
    alpha[FQ - 1, 0] = 1.0
    ar = alpha * np.cos(angi) / TOUT
    ai = -2.0 * np.sin(angi) / TOUT
    ai[0, :] = 0.0
    ai[FQ - 1, :] = 0.0
    ar = np.pad(ar, ((0, 0), (0, TPAD - TOUT)))
    ai = np.pad(ai, ((0, 0), (0, TPAD - TOUT)))

    perm = np.zeros((OB * B, OB * B))
    for b in range(B):
        for o in range(OB):
            perm[b * OB + o, o * B + b] = 1.0
    return (fwd.astype(np.float32), fwdw.astype(np.float32),
            ar.astype(np.float32), ai.astype(np.float32),
            perm.astype(np.float32))


def _matmul_kernel(x_ref, w_ref, o_ref):
    o_ref[...] = jnp.dot(x_ref[...], w_ref[...],
                         preferred_element_type=jnp.float32)


def _mm(x, w, bm, bn):
    m, k = x.shape
    _, n = w.shape
    return pl.pallas_call(
        _matmul_kernel,
        grid=(m // bm, n // bn),
        in_specs=[
            pl.BlockSpec((bm, k), lambda i, j: (i, 0)),
            pl.BlockSpec((k, bn), lambda i, j: (0, j)),
        ],
        out_specs=pl.BlockSpec((bm, bn), lambda i, j: (i, j)),
        out_shape=jax.ShapeDtypeStruct((m, n), jnp.float32),
        compiler_params=pltpu.CompilerParams(
            dimension_semantics=("parallel", "arbitrary"),
        ),
    )(x, w)


def _fused_kernel(w2_ref, ew_ref, x_ref, ar_ref, ai_ref, p_ref, bias_ref,
                  o_ref, wdft_ref, cr_ref, ci_ref):
    s = pl.program_id(0)
    buf = jax.lax.rem(s, 2)
    pbuf = jax.lax.rem(s + 1, 2)

    wdft_ref[:, 0, :] = jnp.dot(w2_ref[...].astype(jnp.bfloat16), ew_ref[...],
                                preferred_element_type=jnp.float32)

    for pj in range(OB // 2):
        for half in range(FQ // FB):
            lo = half * FB
            accs = [None] * 6
            for c in range(CIN):
                xr = x_ref[c, :, lo:lo + FB]
                xi = x_ref[c, :, FQ + lo:FQ + lo + FB]
                xs = xr + xi
                for u in range(2):
                    r = (2 * pj + u) * CIN + c
                    m1 = xr * wdft_ref[r, :, lo:lo + FB]
                    m2 = xi * wdft_ref[r, :, FQ + lo:FQ + lo + FB]
                    m3 = xs * wdft_ref[r, :, 2 * FQ + lo:2 * FQ + lo + FB]
                    j = 3 * u
                    accs[j] = m1 if accs[j] is None else accs[j] + m1
                    accs[j + 1] = m2 if accs[j + 1] is None else accs[j + 1] + m2
                    accs[j + 2] = m3 if accs[j + 2] is None else accs[j + 2] + m3
            for u in range(2):
                a1, a2, a3 = accs[3 * u:3 * u + 3]
                cr_ref[buf, 2 * pj + u, :, lo:lo + FB] = (
                    (a1 - a2).astype(cr_ref.dtype))
                ci_ref[buf, 2 * pj + u, :, lo:lo + FB] = (
                    (a3 - a1 - a2).astype(ci_ref.dtype))

    cr = cr_ref[pbuf].reshape(OB * B, FQ)
    ci = ci_ref[pbuf].reshape(OB * B, FQ)
    crp = jnp.dot(p_ref[...], cr,
                  preferred_element_type=jnp.float32).astype(jnp.bfloat16)
    cip = jnp.dot(p_ref[...], ci,
                  preferred_element_type=jnp.float32).astype(jnp.bfloat16)
    acc = jnp.dot(crp, ar_ref[...], preferred_element_type=jnp.float32)
    acc += jnp.dot(cip, ai_ref[...], preferred_element_type=jnp.float32)
    acc = acc.reshape(B, OB, TPAD) + pltpu.repeat(bias_ref[...], TPAD // 128,
                                                  axis=2)
    o_ref[...] = acc[:, :, :TOUT]


def _fused(w2, ew, xhat, inv_r, inv_i, perm, bias3):
    xv = xhat.reshape(CIN, B, 2 * FQ)
    return pl.pallas_call(
        _fused_kernel,
        grid=(NSTEP + 1,),
        in_specs=[
            pl.BlockSpec((OB * CIN, KW),
                         lambda s: (jnp.minimum(s, NSTEP - 1), 0)),
            pl.BlockSpec((KW, 3 * FQ), lambda s: (0, 0)),
            pl.BlockSpec((CIN, B, 2 * FQ), lambda s: (0, 0, 0)),
            pl.BlockSpec((FQ, TPAD), lambda s: (0, 0)),
            pl.BlockSpec((FQ, TPAD), lambda s: (0, 0)),
            pl.BlockSpec((OB * B, OB * B), lambda s: (0, 0)),
            pl.BlockSpec((B, OB, 128), lambda s: (0, jnp.maximum(s - 1, 0), 0)),
        ],
        out_specs=pl.BlockSpec((B, OB, TOUT),
                               lambda s: (0, jnp.maximum(s - 1, 0), 0)),
        out_shape=jax.ShapeDtypeStruct((B, COUT, TOUT), jnp.float32),
        scratch_shapes=[
            pltpu.VMEM((OB * CIN, 1, 3 * FQ), jnp.float32),
            pltpu.VMEM((2, OB, B, FQ), jnp.bfloat16),
            pltpu.VMEM((2, OB, B, FQ), jnp.bfloat16),
        ],
        compiler_params=pltpu.CompilerParams(
            dimension_semantics=("arbitrary",),
            vmem_limit_bytes=100 * 1024 * 1024,
        ),
    )(w2, ew, xv, inv_r, inv_i, perm, bias3)


def kernel(x, weight, bias):
    fwd, fwdw, inv_r, inv_i, perm = _bases()
    fwd = jnp.asarray(fwd, jnp.bfloat16)
    fwdw = jnp.asarray(fwdw, jnp.bfloat16)
    inv_r = jnp.asarray(inv_r, jnp.bfloat16)
    inv_i = jnp.asarray(inv_i, jnp.bfloat16)
    perm = jnp.asarray(perm, jnp.bfloat16)

    xt = x.transpose(1, 0, 2).reshape(CIN * B, T).astype(jnp.bfloat16)
    xhat = _mm(xt, fwd, bm=512, bn=1024)

    w2 = weight.reshape(COUT * CIN, KW)
    bias3 = jnp.broadcast_to(bias[None, :, None], (B, COUT, 128))

    return _fused(w2, fwdw, xhat, inv_r, inv_i, perm, bias3)
